# baseline (speedup 1.0000x reference)
import functools

import jax
import jax.numpy as jnp
from jax import lax
from jax.experimental import pallas as pl
from jax.experimental.pallas import tpu as pltpu
from jax.experimental.pallas import tpu_sc as plsc

K = 5
THRESHOLD = 0.1


def _select_all_body(cdd_ref, his_ref, w_ref, b_ref, bd_ref, g_ref,
                     idx_out, wgt_out):
    x = cdd_ref[...]
    h = his_ref[...]
    wm = w_ref[...]
    bias = b_ref[...]

    contract_last = (((1,), (1,)), ((), ()))
    xp = jax.lax.dot_general(x, wm, contract_last,
                             preferred_element_type=jnp.float32) + bias
    hp = jax.lax.dot_general(h, wm, contract_last,
                             preferred_element_type=jnp.float32) + bias
    xn = xp / jnp.maximum(
        jnp.sqrt(jnp.sum(xp * xp, axis=1, keepdims=True)), 1e-12)
    hn = hp / jnp.maximum(
        jnp.sqrt(jnp.sum(hp * hp, axis=1, keepdims=True)), 1e-12)

    big = jax.lax.dot_general(xn, hn, contract_last,
                              preferred_element_type=jnp.float32)
    big = big * bd_ref[...]
    attn = jax.lax.dot_general(big, g_ref[...], (((1,), (0,)), ((), ())),
                               preferred_element_type=jnp.float32,
                               precision=jax.lax.Precision.HIGHEST)

    bc_dim, h_dim = attn.shape
    c_dim = idx_out.shape[1]
    iota_h = jax.lax.broadcasted_iota(jnp.int32, (bc_dim, h_dim), 1)
    row_base = (jax.lax.broadcasted_iota(jnp.int32, (bc_dim, 1), 0)
                // c_dim) * h_dim

    a = attn
    vals_cols, idx_cols = [], []
    for _ in range(K):
        m = jnp.max(a, axis=1, keepdims=True)
        picked = jnp.min(jnp.where(a == m, iota_h, h_dim), axis=1,
                         keepdims=True)
        vals_cols.append(m)
        idx_cols.append(picked + row_base)
        a = jnp.where(iota_h == picked, -jnp.inf, a)
    vals = jnp.concatenate(vals_cols, axis=1)
    idx_out[...] = jnp.concatenate(idx_cols, axis=1)
    wgt_out[...] = jnp.where(vals < THRESHOLD, 0.0, vals)


def _select_body(cdd_ref, his_ref, w_ref, b_ref, hm_ref, idx_out, wgt_out,
                 msk_out):
    x = cdd_ref[0]
    h = his_ref[0]
    wm = w_ref[...]
    bias = b_ref[0]

    contract_last = (((1,), (1,)), ((), ()))
    xp = jax.lax.dot_general(x, wm, contract_last,
                             preferred_element_type=jnp.float32) + bias[None, :]
    hp = jax.lax.dot_general(h, wm, contract_last,
                             preferred_element_type=jnp.float32) + bias[None, :]
    xn = xp / jnp.maximum(
        jnp.sqrt(jnp.sum(xp * xp, axis=1, keepdims=True)), 1e-12)
    hn = hp / jnp.maximum(
        jnp.sqrt(jnp.sum(hp * hp, axis=1, keepdims=True)), 1e-12)
    attn = jax.lax.dot_general(xn, hn, contract_last,
                               preferred_element_type=jnp.float32)

    c_dim, h_dim = attn.shape
    iota_h = jax.lax.broadcasted_iota(jnp.int32, (c_dim, h_dim), 1)
    a = attn
    vals_cols = []
    idx_cols = []
    for _ in range(K):
        m = jnp.max(a, axis=1, keepdims=True)
        picked = jnp.min(jnp.where(a == m, iota_h, h_dim), axis=1,
                         keepdims=True)
        vals_cols.append(m)
        idx_cols.append(picked)
        a = jnp.where(iota_h == picked, -jnp.inf, a)
    vals = jnp.concatenate(vals_cols, axis=1)
    idx = jnp.concatenate(idx_cols, axis=1)
    wgt = jnp.where(vals < THRESHOLD, 0.0, vals)

    idx_out[...] = (idx + pl.program_id(0) * h_dim)[None]
    wgt_out[...] = wgt[None]

    hm = hm_ref[0]
    msk_cols = []
    for picked in idx_cols:
        onehot = jnp.where(iota_h == picked, 1.0, 0.0)
        m_k = jax.lax.dot_general(onehot, hm, (((1,), (0,)), ((), ())),
                                  preferred_element_type=jnp.float32)
        msk_cols.append(m_k[:, None, :])
    msk_out[...] = jnp.concatenate(msk_cols, axis=1)[None]


def _make_sc_gather(n_rows, sub, d_dim, chunk, n_chunks, n_workers):
    mesh = plsc.VectorSubcoreMesh(core_axis_name="c", subcore_axis_name="s")
    lanes = 16
    dsteps = d_dim // lanes

    @functools.partial(
        pl.kernel,
        mesh=mesh,
        out_type=[
            jax.ShapeDtypeStruct((n_rows, sub, d_dim), jnp.float32),
            jax.ShapeDtypeStruct((n_rows, d_dim), jnp.float32),
        ],
        scratch_types=[
            pltpu.VMEM((chunk,), jnp.int32),
            pltpu.VMEM((chunk,), jnp.int32),
            pltpu.VMEM((chunk, lanes), jnp.float32),
            pltpu.VMEM((chunk, lanes), jnp.float32),
            pltpu.VMEM((chunk, sub, d_dim), jnp.float32),
            pltpu.VMEM((chunk, sub, d_dim), jnp.float32),
            pltpu.VMEM((chunk, d_dim), jnp.float32),
            pltpu.VMEM((chunk, d_dim), jnp.float32),
            pltpu.SemaphoreType.DMA,
            pltpu.SemaphoreType.DMA,
            pltpu.SemaphoreType.DMA,
            pltpu.SemaphoreType.DMA,
        ],
    )
    def sc_gather(idx_hbm, w_hbm, table_hbm, hm_hbm, out_he_hbm, out_hm_hbm,
                  idx_v0, idx_v1, w_v0, w_v1, rows_v0, rows_v1,
                  mask_v0, mask_v1, sem0, sem1, msem0, msem1):
        wid = lax.axis_index("s") * 2 + lax.axis_index("c")
        bufs = [(idx_v0, w_v0, rows_v0, mask_v0, sem0, msem0),
                (idx_v1, w_v1, rows_v1, mask_v1, sem1, msem1)]
        max_t = (n_chunks + n_workers - 1) // n_workers

        def fire(g, buf):
            idx_v, w_v, rows_v, mask_v, sem, msem = buf
            pltpu.sync_copy(idx_hbm.at[g], idx_v)
            pltpu.sync_copy(w_hbm.at[g], w_v)
            pltpu.async_copy(table_hbm.at[idx_v], rows_v, sem)
            pltpu.async_copy(hm_hbm.at[idx_v], mask_v, msem)

        def process(g, buf):
            idx_v, w_v, rows_v, mask_v, sem, msem = buf
            pltpu.make_async_copy(table_hbm.at[idx_v], rows_v, sem).wait()
            wsplat = [w_v[r, pl.ds(0, lanes)] for r in range(chunk)]

            def scale_step(s, carry):
                for r in range(chunk):
                    for j in range(dsteps):
                        sl = pl.ds(j * lanes, lanes)
                        rows_v[r, s, sl] = rows_v[r, s, sl] * wsplat[r]
                return carry

            lax.fori_loop(0, sub, scale_step, 0)
            pltpu.sync_copy(rows_v, out_he_hbm.at[pl.ds(g * chunk, chunk)])
            pltpu.make_async_copy(hm_hbm.at[idx_v], mask_v, msem).wait()
            pltpu.sync_copy(mask_v, out_hm_hbm.at[pl.ds(g * chunk, chunk)])

        for t in range(max_t):
            g = wid + t * n_workers
            pl.when(g < n_chunks)(lambda: fire(g, bufs[t % 2]))
            if t >= 1:
                gp = wid + (t - 1) * n_workers
                pl.when(gp < n_chunks)(
                    lambda: process(gp, bufs[(t - 1) % 2]))
        g_last = wid + (max_t - 1) * n_workers
        pl.when(g_last < n_chunks)(
            lambda: process(g_last, bufs[(max_t - 1) % 2]))

    return sc_gather


def kernel(cdd_repr, his_repr, his_embedding, his_attn_mask, W, b):
    B, C, D = cdd_repr.shape
    H = his_repr.shape[1]
    S = his_attn_mask.shape[2]
    L = his_embedding.shape[3]
    CK = C * K

    BC, BH = B * C, B * H
    cdd2 = cdd_repr.reshape(BC, D)
    his2 = his_repr.reshape(BH, D)
    hm2 = his_attn_mask.reshape(BH, S)
    bi = jnp.arange(BC, dtype=jnp.int32) // C
    bj = jnp.arange(BH, dtype=jnp.int32) // H
    bd = (bi[:, None] == bj[None, :]).astype(jnp.float32)
    g = (jnp.arange(BH, dtype=jnp.int32)[:, None] % H
         == jnp.arange(H, dtype=jnp.int32)[None, :]).astype(jnp.float32)

    whole = lambda shape: pl.BlockSpec(shape, lambda: tuple(0 for _ in shape))
    idx, wgt = pl.pallas_call(
        _select_all_body,
        in_specs=[
            whole((BC, D)),
            whole((BH, D)),
            whole((D, D)),
            whole((1, D)),
            whole((BC, BH)),
            whole((BH, H)),
        ],
        out_specs=[whole((BC, K)), whole((BC, K))],
        out_shape=[
            jax.ShapeDtypeStruct((BC, K), jnp.int32),
            jax.ShapeDtypeStruct((BC, K), jnp.float32),
        ],
    )(cdd2, his2, W, b.reshape(1, D), bd, g)

    n_rows = B * CK
    chunk = 4
    n_chunks = n_rows // chunk
    idx_g = idx.reshape(n_chunks, chunk)
    wgt_g = jnp.broadcast_to(wgt.reshape(n_rows, 1),
                             (n_rows, 16)).reshape(n_chunks, chunk, 16)
    sub = S * L * D // 128
    table = his_embedding.reshape(B * H, sub, 128)
    hm128 = jnp.pad(hm2, ((0, 0), (0, 128 - S)))

    sc_gather = _make_sc_gather(n_rows, sub, 128, chunk, n_chunks, 32)
    out_he, out_hm = sc_gather(idx_g, wgt_g, table, hm128)

    his_activated = out_he.reshape(B, C, K, S, L, D)
    his_mask_activated = out_hm[:, :S].reshape(B, C, K, S)
    return (his_activated, his_mask_activated)


# speedup vs baseline: 1.0749x; 1.0749x over previous
import functools

import jax
import jax.numpy as jnp
from jax import lax
from jax.experimental import pallas as pl
from jax.experimental.pallas import tpu as pltpu
from jax.experimental.pallas import tpu_sc as plsc

K = 5
THRESHOLD = 0.1


def _select_all_body(cdd_ref, his_ref, w_ref, b_ref, bd_ref, g_ref, hm_ref,
                     idx_out, wgt_out, *msk_outs):
    x = cdd_ref[...]
    h = his_ref[...]
    wm = w_ref[...]
    bias = b_ref[...]

    contract_last = (((1,), (1,)), ((), ()))
    xp = jax.lax.dot_general(x, wm, contract_last,
                             preferred_element_type=jnp.float32) + bias
    hp = jax.lax.dot_general(h, wm, contract_last,
                             preferred_element_type=jnp.float32) + bias
    xn = xp / jnp.maximum(
        jnp.sqrt(jnp.sum(xp * xp, axis=1, keepdims=True)), 1e-12)
    hn = hp / jnp.maximum(
        jnp.sqrt(jnp.sum(hp * hp, axis=1, keepdims=True)), 1e-12)

    big = jax.lax.dot_general(xn, hn, contract_last,
                              preferred_element_type=jnp.float32)
    big = big * bd_ref[...]
    attn = jax.lax.dot_general(big, g_ref[...], (((1,), (0,)), ((), ())),
                               preferred_element_type=jnp.float32,
                               precision=jax.lax.Precision.HIGHEST)

    bc_dim, h_dim = attn.shape
    c_dim = idx_out.shape[1]
    iota_h = jax.lax.broadcasted_iota(jnp.int32, (bc_dim, h_dim), 1)
    row_base = (jax.lax.broadcasted_iota(jnp.int32, (bc_dim, 1), 0)
                // c_dim) * h_dim

    hm = hm_ref[...]
    a = attn
    vals_cols, idx_cols = [], []
    for k in range(K):
        m = jnp.max(a, axis=1, keepdims=True)
        picked = jnp.min(jnp.where(a == m, iota_h, h_dim), axis=1,
                         keepdims=True)
        vals_cols.append(m)
        g_idx = picked + row_base
        idx_cols.append(g_idx)
        a = jnp.where(iota_h == picked, -jnp.inf, a)
        iota_bh = jax.lax.broadcasted_iota(jnp.int32, (bc_dim, hm.shape[0]),
                                           1)
        onehot = jnp.where(iota_bh == g_idx, 1.0, 0.0)
        msk_outs[k][...] = jax.lax.dot_general(
            onehot, hm, (((1,), (0,)), ((), ())),
            preferred_element_type=jnp.float32,
            precision=jax.lax.Precision.HIGHEST)
    vals = jnp.concatenate(vals_cols, axis=1)
    idx_out[...] = jnp.concatenate(idx_cols, axis=1)
    wgt = jnp.where(vals < THRESHOLD, 0.0, vals)
    wgt_out[...] = jnp.broadcast_to(wgt[:, :, None], wgt_out.shape)


def _make_sc_gather(n_rows, sub, d_dim, chunk, n_chunks, n_workers):
    mesh = plsc.VectorSubcoreMesh(core_axis_name="c", subcore_axis_name="s")
    lanes = 16
    dsteps = d_dim // lanes

    @functools.partial(
        pl.kernel,
        mesh=mesh,
        out_type=jax.ShapeDtypeStruct((n_rows, sub, d_dim), jnp.float32),
        scratch_types=[
            pltpu.VMEM((chunk,), jnp.int32),
            pltpu.VMEM((chunk,), jnp.int32),
            pltpu.VMEM((chunk, lanes), jnp.float32),
            pltpu.VMEM((chunk, lanes), jnp.float32),
            pltpu.VMEM((chunk, sub, d_dim), jnp.float32),
            pltpu.VMEM((chunk, sub, d_dim), jnp.float32),
            pltpu.SemaphoreType.DMA,
            pltpu.SemaphoreType.DMA,
        ],
    )
    def sc_gather(idx_hbm, w_hbm, table_hbm, out_he_hbm,
                  idx_v0, idx_v1, w_v0, w_v1, rows_v0, rows_v1,
                  sem0, sem1):
        wid = lax.axis_index("s") * 2 + lax.axis_index("c")
        bufs = [(idx_v0, w_v0, rows_v0, sem0),
                (idx_v1, w_v1, rows_v1, sem1)]
        max_t = (n_chunks + n_workers - 1) // n_workers

        def fire(g, buf):
            idx_v, w_v, rows_v, sem = buf
            pltpu.sync_copy(idx_hbm.at[g], idx_v)
            pltpu.sync_copy(w_hbm.at[g], w_v)
            pltpu.async_copy(table_hbm.at[idx_v], rows_v, sem)

        def process(g, buf):
            idx_v, w_v, rows_v, sem = buf
            pltpu.make_async_copy(table_hbm.at[idx_v], rows_v, sem).wait()
            wsplat = [w_v[r, pl.ds(0, lanes)] for r in range(chunk)]

            def scale_step(s, carry):
                for r in range(chunk):
                    for j in range(dsteps):
                        sl = pl.ds(j * lanes, lanes)
                        rows_v[r, s, sl] = rows_v[r, s, sl] * wsplat[r]
                return carry

            lax.fori_loop(0, sub, scale_step, 0)
            pltpu.sync_copy(rows_v, out_he_hbm.at[pl.ds(g * chunk, chunk)])

        for t in range(max_t):
            g = wid + t * n_workers
            pl.when(g < n_chunks)(lambda: fire(g, bufs[t % 2]))
            if t >= 1:
                gp = wid + (t - 1) * n_workers
                pl.when(gp < n_chunks)(
                    lambda: process(gp, bufs[(t - 1) % 2]))
        g_last = wid + (max_t - 1) * n_workers
        pl.when(g_last < n_chunks)(
            lambda: process(g_last, bufs[(max_t - 1) % 2]))

    return sc_gather


def kernel(cdd_repr, his_repr, his_embedding, his_attn_mask, W, b):
    B, C, D = cdd_repr.shape
    H = his_repr.shape[1]
    S = his_attn_mask.shape[2]
    L = his_embedding.shape[3]
    CK = C * K

    BC, BH = B * C, B * H
    cdd2 = cdd_repr.reshape(BC, D)
    his2 = his_repr.reshape(BH, D)
    hm2 = his_attn_mask.reshape(BH, S)
    bi = jnp.arange(BC, dtype=jnp.int32) // C
    bj = jnp.arange(BH, dtype=jnp.int32) // H
    bd = (bi[:, None] == bj[None, :]).astype(jnp.float32)
    g = (jnp.arange(BH, dtype=jnp.int32)[:, None] % H
         == jnp.arange(H, dtype=jnp.int32)[None, :]).astype(jnp.float32)

    whole = lambda shape: pl.BlockSpec(shape, lambda: tuple(0 for _ in shape))
    outs = pl.pallas_call(
        _select_all_body,
        in_specs=[
            whole((BC, D)),
            whole((BH, D)),
            whole((D, D)),
            whole((1, D)),
            whole((BC, BH)),
            whole((BH, H)),
            whole((BH, S)),
        ],
        out_specs=[whole((BC, K)), whole((BC, K, 16))]
        + [whole((BC, S))] * K,
        out_shape=[
            jax.ShapeDtypeStruct((BC, K), jnp.int32),
            jax.ShapeDtypeStruct((BC, K, 16), jnp.float32),
        ] + [jax.ShapeDtypeStruct((BC, S), jnp.float32)] * K,
    )(cdd2, his2, W, b.reshape(1, D), bd, g, hm2)
    idx_g, wgt_g = outs[0], outs[1]
    msk = jnp.stack(outs[2:], axis=1).reshape(B, C, K, S)

    n_rows = B * CK
    chunk = K
    n_chunks = BC
    sub = S * L * D // 128
    table = his_embedding.reshape(B * H, sub, 128)

    sc_gather = _make_sc_gather(n_rows, sub, 128, chunk, n_chunks, 32)
    out_he = sc_gather(idx_g, wgt_g, table)

    his_activated = out_he.reshape(B, C, K, S, L, D)
    return (his_activated, msk)
